# baseline (speedup 1.0000x reference)
import functools

import jax
import jax.numpy as jnp
from jax import lax
from jax.experimental import pallas as pl
from jax.experimental.pallas import tpu as pltpu
from jax.experimental.pallas import tpu_sc as plsc

_L = 16
_N = 64
_ROW = 1024
_DIM = 0


def _scatter_body(bt_hbm, x_hbm, y_hbm, b_hbm, x_v, y_v, idx_v, val_v, bt_v, sem):
    cid = lax.axis_index("c")
    sid = lax.axis_index("s")

    @pl.when(jnp.logical_and(cid == 0, sid == 0))
    def _():
        pltpu.sync_copy(x_hbm, x_v)
        pltpu.sync_copy(y_hbm, y_v)
        pltpu.sync_copy(bt_hbm, bt_v)
        for j in range(_N // _L):
            s = pl.ds(j * _L, _L)
            idx_v[s] = x_v[s] * _ROW + y_v[s]
        pltpu.async_copy(b_hbm.at[idx_v], val_v, sem).wait()
        for j in range(_N // _L):
            s = pl.ds(j * _L, _L)
            val_v[s] = val_v[s] + bt_v[s]
        pltpu.async_copy(val_v, b_hbm.at[idx_v], sem).wait()


_scatter = pl.kernel(
    _scatter_body,
    out_type=(),
    mesh=plsc.VectorSubcoreMesh(core_axis_name="c", subcore_axis_name="s"),
    scratch_types=[
        pltpu.VMEM((_N,), jnp.int32),
        pltpu.VMEM((_N,), jnp.int32),
        pltpu.VMEM((_N,), jnp.int32),
        pltpu.VMEM((_N,), jnp.float32),
        pltpu.VMEM((_N,), jnp.float32),
        pltpu.SemaphoreType.DMA,
    ],
)


def kernel(B, Bt, x, y):
    shape = B.shape
    b_ref = jax.new_ref(B.reshape(-1))
    _scatter(Bt, x + jnp.int32(_DIM * shape[1]), y, b_ref)
    return b_ref[...].reshape(shape)
